# Initial kernel scaffold; baseline (speedup 1.0000x reference)
import jax
import jax.numpy as jnp
from jax.experimental import pallas as pl


def kernel(center, outside, negative, W_center, W_outside):
    raise NotImplementedError("write your pallas kernel here")


# speedup vs baseline: 4.5988x; 4.5988x over previous
import functools

import jax
import jax.numpy as jnp
from jax import lax
from jax.experimental import pallas as pl
from jax.experimental.pallas import tpu as pltpu
from jax.experimental.pallas import tpu_sc as plsc

B = 16384
K = 20
EMB = 64

_NC = 2
_NS = 16
_L = 16
_NW = _NC * _NS
_BPW = B // _NW
_CROWS = _BPW // 128
_NROWS = _BPW * K // 128
_NDOTS = _BPW * (K + 1)

_mesh = plsc.VectorSubcoreMesh(core_axis_name="c", subcore_axis_name="s")


@functools.partial(
    pl.kernel,
    out_type=jax.ShapeDtypeStruct((_NW, _L), jnp.float32),
    mesh=_mesh,
    compiler_params=pltpu.CompilerParams(
        needs_layout_passes=False, use_tc_tiling_on_sc=False),
    scratch_types=[
        pltpu.VMEM((_CROWS, 128), jnp.int32),
        pltpu.VMEM((_CROWS, 128), jnp.int32),
        pltpu.VMEM((_NROWS, 128), jnp.int32),
        pltpu.VMEM((_BPW, EMB), jnp.float32),
        pltpu.VMEM((_BPW, EMB), jnp.float32),
        pltpu.VMEM((128, EMB), jnp.float32),
        pltpu.VMEM((_NDOTS,), jnp.float32),
        pltpu.VMEM((_L,), jnp.float32),
        pltpu.SemaphoreType.DMA,
    ],
)
def _sc_loss(cidx_hbm, oidx_hbm, nidx_hbm, wc_hbm, wo_hbm, out_hbm,
             cidx_v, oidx_v, nidx_v, c_all, o_all, nbuf, dots, acc_v, sem):
    wid = lax.axis_index("s") * _NC + lax.axis_index("c")

    pltpu.sync_copy(cidx_hbm.at[pl.ds(wid * _CROWS, _CROWS)], cidx_v)
    pltpu.sync_copy(oidx_hbm.at[pl.ds(wid * _CROWS, _CROWS)], oidx_v)
    pltpu.sync_copy(nidx_hbm.at[pl.ds(wid * _NROWS, _NROWS)], nidx_v)

    copies = []
    for blk in range(_CROWS):
        copies.append(pltpu.async_copy(
            wc_hbm.at[cidx_v.at[blk]], c_all.at[pl.ds(blk * 128, 128)], sem))
        copies.append(pltpu.async_copy(
            wo_hbm.at[oidx_v.at[blk]], o_all.at[pl.ds(blk * 128, 128)], sem))
    for cp in copies:
        cp.wait()

    lanes = lax.iota(jnp.int32, _L)
    m15 = lanes == _L - 1

    def uovc_body(b, carry):
        d = c_all[b, pl.ds(0, _L)] * o_all[b, pl.ds(0, _L)]
        for q in range(1, EMB // _L):
            d = d + c_all[b, pl.ds(q * _L, _L)] * o_all[b, pl.ds(q * _L, _L)]
        cum = plsc.cumsum(d)
        tgt = jnp.full((_L,), K * _BPW + b, jnp.int32)
        plsc.store_scatter(dots, [tgt], cum, mask=m15)
        return carry

    lax.fori_loop(0, _BPW, uovc_body, 0)

    def nrow_body(r, carry):
        pltpu.async_copy(wo_hbm.at[nidx_v.at[r]], nbuf, sem).wait()

        def dot_body(j, c2):
            flat = r * 128 + j
            b = flat // K
            d = c_all[b, pl.ds(0, _L)] * nbuf[j, pl.ds(0, _L)]
            for q in range(1, EMB // _L):
                d = d + c_all[b, pl.ds(q * _L, _L)] * nbuf[j, pl.ds(q * _L, _L)]
            cum = plsc.cumsum(d)
            tgt = jnp.full((_L,), flat, jnp.int32)
            plsc.store_scatter(dots, [tgt], -cum, mask=m15)
            return c2

        lax.fori_loop(0, 128, dot_body, 0)
        return carry

    lax.fori_loop(0, _NROWS, nrow_body, 0)

    def red_body(t, acc):
        x = dots[pl.ds(pl.multiple_of(t * _L, _L), _L)]
        e = jnp.exp(-jnp.abs(x))
        z = e / (e + 2.0)
        z2 = z * z
        log1p = z * (2.0 + z2 * (2.0 / 3.0 + z2 * (2.0 / 5.0 + z2 * (2.0 / 7.0))))
        return acc + (jnp.minimum(x, 0.0) - log1p)

    acc = lax.fori_loop(0, _NDOTS // _L, red_body, jnp.zeros((_L,), jnp.float32))
    acc_v[...] = acc
    pltpu.sync_copy(acc_v, out_hbm.at[wid])


def _finish_body(p_ref, o_ref):
    o_ref[...] = (-jnp.sum(p_ref[...]) * (1.0 / B)).reshape(1, 1)


_finish = pl.pallas_call(
    _finish_body,
    out_shape=jax.ShapeDtypeStruct((1, 1), jnp.float32),
)


def kernel(center, outside, negative, W_center, W_outside):
    cidx = center.astype(jnp.int32).reshape(B // 128, 128)
    oidx = outside.astype(jnp.int32).reshape(B // 128, 128)
    nidx = negative.astype(jnp.int32).reshape(B * K // 128, 128)
    partials = _sc_loss(cidx, oidx, nidx, W_center, W_outside)
    return _finish(partials)[0, 0]


# speedup vs baseline: 5.0818x; 1.1050x over previous
import functools

import jax
import jax.numpy as jnp
from jax import lax
from jax.experimental import pallas as pl
from jax.experimental.pallas import tpu as pltpu
from jax.experimental.pallas import tpu_sc as plsc

VOC = 1000000
B = 16384
K = 20
EMB = 64

_NW = 32
_L = 16
_BPW = B // _NW
_CBLK = _BPW // 128
_NROWS = _BPW * K // 128
_Q = EMB // _L

_mesh = plsc.VectorSubcoreMesh(core_axis_name="c", subcore_axis_name="s")


@functools.partial(
    pl.kernel,
    out_type=jax.ShapeDtypeStruct((_NW * _L,), jnp.float32),
    mesh=_mesh,
    compiler_params=pltpu.CompilerParams(needs_layout_passes=False),
    scratch_types=[
        pltpu.VMEM((8, 128), jnp.int32),
        pltpu.VMEM((_NROWS, 128), jnp.int32),
        pltpu.VMEM((_BPW * K,), jnp.int32),
        pltpu.VMEM((_BPW,), jnp.int32),
        pltpu.VMEM((_BPW, 128), jnp.float32),
        pltpu.VMEM((128, 128), jnp.float32),
        pltpu.VMEM((128, 128), jnp.float32),
        pltpu.VMEM((128,), jnp.float32),
        pltpu.VMEM((_L,), jnp.float32),
        pltpu.SemaphoreType.DMA,
        pltpu.SemaphoreType.DMA,
    ],
)
def _sc_loss(gco_hbm, gn_hbm, pkn_hbm, pou_hbm, wc2_hbm, wo2_hbm, out_hbm,
             idx_v, gn_v, pkn_v, pou_v, c_all, bufa, bufb, dblk, acc_v,
             sema, semb):
    wid = lax.axis_index("s") * 2 + lax.axis_index("c")

    pltpu.sync_copy(gco_hbm.at[pl.ds(wid * 8, 8)], idx_v)
    pltpu.sync_copy(gn_hbm.at[pl.ds(wid * _NROWS, _NROWS)], gn_v)
    pltpu.sync_copy(pkn_hbm.at[pl.ds(wid * _BPW * K, _BPW * K)], pkn_v)
    pltpu.sync_copy(pou_hbm.at[pl.ds(wid * _BPW, _BPW)], pou_v)

    iota = lax.iota(jnp.int32, _L)
    m15 = iota == _L - 1
    acc_v[...] = jnp.zeros((_L,), jnp.float32)

    cps = [
        pltpu.async_copy(wc2_hbm.at[idx_v.at[blk]],
                         c_all.at[pl.ds(blk * 128, 128)], sema)
        for blk in range(_CBLK)
    ]
    for cp in cps:
        cp.wait()

    def accum_dblk():
        def rb(t, a):
            x = dblk[pl.ds(pl.multiple_of(t * _L, _L), _L)]
            e = jnp.exp(-jnp.abs(x))
            z = e / (e + 2.0)
            z2 = z * z
            l1p = z * (2.0 + z2 * (2.0 / 3.0 + z2 * (2.0 / 5.0 + z2 * (2.0 / 7.0))))
            return a + (jnp.minimum(x, 0.0) - l1p)

        acc_v[...] = lax.fori_loop(0, 128 // _L, rb, acc_v[...])

    bufs = (bufa, bufb)
    sems = (sema, semb)

    pltpu.async_copy(wo2_hbm.at[idx_v.at[_CBLK]], bufa, sema)
    for blk in range(_CBLK):
        buf, sem = bufs[blk % 2], sems[blk % 2]
        pltpu.make_async_copy(wo2_hbm.at[idx_v.at[_CBLK + blk]], buf, sem).wait()
        if blk + 1 < _CBLK:
            pltpu.async_copy(wo2_hbm.at[idx_v.at[_CBLK + blk + 1]],
                             bufs[(blk + 1) % 2], sems[(blk + 1) % 2])

        @plsc.parallel_loop(0, 128, unroll=4)
        def dot_o(j, _blk=blk, _buf=buf):
            b = _blk * 128 + j
            jv = jnp.full((_L,), j, jnp.int32)
            pv = plsc.load_gather(pou_v, [jnp.full((_L,), b, jnp.int32)])
            offc = (pv & 1) << 6
            offo = (pv & 2) << 5
            bv = jnp.full((_L,), b, jnp.int32)
            d = jnp.zeros((_L,), jnp.float32)
            for q in range(_Q):
                cq = plsc.load_gather(c_all, [bv, offc + (iota + q * _L)])
                oq = plsc.load_gather(_buf, [jv, offo + (iota + q * _L)])
                d = d + cq * oq
            cum = plsc.cumsum(d)
            plsc.store_scatter(dblk, [jv], cum, mask=m15)

        accum_dblk()

    pltpu.async_copy(wo2_hbm.at[gn_v.at[0]], bufa, sema)

    def pair_body(i, carry):
        rr = i * 2
        for pb in range(2):
            r = rr + pb
            buf, sem = bufs[pb], sems[pb]
            pltpu.make_async_copy(wo2_hbm.at[gn_v.at[r]], buf, sem).wait()

            @pl.when(r + 1 < _NROWS)
            def _():
                pltpu.async_copy(wo2_hbm.at[gn_v.at[r + 1]],
                                 bufs[(pb + 1) % 2], sems[(pb + 1) % 2])

            @plsc.parallel_loop(0, 128, unroll=4)
            def dot_n(j, _buf=buf, _r=r):
                flat = _r * 128 + j
                jv = jnp.full((_L,), j, jnp.int32)
                pv = plsc.load_gather(pkn_v, [jnp.full((_L,), flat, jnp.int32)])
                bv = lax.shift_right_logical(pv, 2)
                offc = (pv & 1) << 6
                offx = (pv & 2) << 5
                d = jnp.zeros((_L,), jnp.float32)
                for q in range(_Q):
                    cq = plsc.load_gather(c_all, [bv, offc + (iota + q * _L)])
                    xq = plsc.load_gather(_buf, [jv, offx + (iota + q * _L)])
                    d = d + cq * xq
                cum = plsc.cumsum(d)
                plsc.store_scatter(dblk, [jv], -cum, mask=m15)

            accum_dblk()
        return carry

    lax.fori_loop(0, _NROWS // 2, pair_body, 0)

    pltpu.sync_copy(acc_v, out_hbm.at[pl.ds(wid * _L, _L)])


def _finish_body(p_ref, o_ref):
    o_ref[...] = (-jnp.sum(p_ref[...]) * (1.0 / B)).reshape(1, 1)


_finish = pl.pallas_call(
    _finish_body,
    out_shape=jax.ShapeDtypeStruct((1, 1), jnp.float32),
)


def kernel(center, outside, negative, W_center, W_outside):
    c_i = center.astype(jnp.int32).reshape(B)
    o_i = outside.astype(jnp.int32).reshape(B)
    n_i = negative.astype(jnp.int32).reshape(B * K)

    gc = (c_i >> 1).reshape(_NW, _CBLK, 128)
    go = (o_i >> 1).reshape(_NW, _CBLK, 128)
    gco = jnp.concatenate([gc, go], axis=1).reshape(_NW * 8, 128)
    gn = (n_i >> 1).reshape(B * K // 128, 128)

    cpar = c_i & 1
    bloc = (jnp.arange(B * K, dtype=jnp.int32) // K) % _BPW
    pkn = bloc * 4 + (n_i & 1) * 2 + jnp.repeat(cpar, K)
    pou = (o_i & 1) * 2 + cpar

    wc2 = W_center.reshape(VOC // 2, 128)
    wo2 = W_outside.reshape(VOC // 2, 128)

    partials = _sc_loss(gco, gn, pkn, pou, wc2, wo2)
    return _finish(partials.reshape(_NW, _L))[0, 0]


# speedup vs baseline: 5.2761x; 1.0382x over previous
import functools

import jax
import jax.numpy as jnp
from jax import lax
from jax.experimental import pallas as pl
from jax.experimental.pallas import tpu as pltpu
from jax.experimental.pallas import tpu_sc as plsc

VOC = 1000000
B = 16384
K = 20
EMB = 64

_NW = 32
_L = 16
_BPW = B // _NW
_CBLK = _BPW // 128
_NROWS = _BPW * K // 128
_Q = EMB // _L

_mesh = plsc.VectorSubcoreMesh(core_axis_name="c", subcore_axis_name="s")


@functools.partial(
    pl.kernel,
    out_type=jax.ShapeDtypeStruct((_NW * _L,), jnp.float32),
    mesh=_mesh,
    compiler_params=pltpu.CompilerParams(
        needs_layout_passes=False, use_tc_tiling_on_sc=False),
    scratch_types=[
        pltpu.VMEM((8, 128), jnp.int32),
        pltpu.VMEM((_NROWS, 128), jnp.int32),
        pltpu.VMEM((_BPW * K,), jnp.int32),
        pltpu.VMEM((_BPW, EMB), jnp.float32),
        pltpu.VMEM((128, EMB), jnp.float32),
        pltpu.VMEM((128, EMB), jnp.float32),
        pltpu.VMEM((128,), jnp.float32),
        pltpu.VMEM((_L,), jnp.float32),
        pltpu.SemaphoreType.DMA,
        pltpu.SemaphoreType.DMA,
    ],
)
def _sc_loss(gco_hbm, gn_hbm, pkn_hbm, wc_hbm, wo_hbm, out_hbm,
             idx_v, gn_v, pkn_v, c_all, bufa, bufb, dblk, acc_v,
             sema, semb):
    wid = lax.axis_index("s") * 2 + lax.axis_index("c")

    pltpu.sync_copy(gco_hbm.at[pl.ds(wid * 8, 8)], idx_v)
    pltpu.sync_copy(gn_hbm.at[pl.ds(wid * _NROWS, _NROWS)], gn_v)
    pltpu.sync_copy(pkn_hbm.at[pl.ds(wid * _BPW * K, _BPW * K)], pkn_v)

    iota = lax.iota(jnp.int32, _L)
    m15 = iota == _L - 1
    acc_v[...] = jnp.zeros((_L,), jnp.float32)

    cps = [
        pltpu.async_copy(wc_hbm.at[idx_v.at[blk]],
                         c_all.at[pl.ds(blk * 128, 128)], sema)
        for blk in range(_CBLK)
    ]
    for cp in cps:
        cp.wait()

    def accum_dblk():
        def rb(t, a):
            x = dblk[pl.ds(pl.multiple_of(t * _L, _L), _L)]
            e = jnp.exp(-jnp.abs(x))
            z = e / (e + 2.0)
            z2 = z * z
            l1p = z * (2.0 + z2 * (2.0 / 3.0 + z2 * (2.0 / 5.0 + z2 * (2.0 / 7.0))))
            return a + (jnp.minimum(x, 0.0) - l1p)

        acc_v[...] = lax.fori_loop(0, 128 // _L, rb, acc_v[...])

    bufs = (bufa, bufb)
    sems = (sema, semb)

    pltpu.async_copy(wo_hbm.at[idx_v.at[_CBLK]], bufa, sema)
    for blk in range(_CBLK):
        buf, sem = bufs[blk % 2], sems[blk % 2]
        pltpu.make_async_copy(wo_hbm.at[idx_v.at[_CBLK + blk]], buf, sem).wait()
        if blk + 1 < _CBLK:
            pltpu.async_copy(wo_hbm.at[idx_v.at[_CBLK + blk + 1]],
                             bufs[(blk + 1) % 2], sems[(blk + 1) % 2])

        @plsc.parallel_loop(0, 128, unroll=4)
        def dot_o(j, _blk=blk, _buf=buf):
            b = _blk * 128 + j
            jv = jnp.full((_L,), j, jnp.int32)
            d = jnp.zeros((_L,), jnp.float32)
            for q in range(_Q):
                d = d + (c_all[b, pl.ds(q * _L, _L)] * _buf[j, pl.ds(q * _L, _L)])
            cum = plsc.cumsum(d)
            plsc.store_scatter(dblk, [jv], cum, mask=m15)

        accum_dblk()

    pltpu.async_copy(wo_hbm.at[gn_v.at[0]], bufa, sema)

    def pair_body(i, carry):
        rr = i * 2
        for pb in range(2):
            r = rr + pb
            buf, sem = bufs[pb], sems[pb]
            pltpu.make_async_copy(wo_hbm.at[gn_v.at[r]], buf, sem).wait()

            @pl.when(r + 1 < _NROWS)
            def _():
                pltpu.async_copy(wo_hbm.at[gn_v.at[r + 1]],
                                 bufs[(pb + 1) % 2], sems[(pb + 1) % 2])

            @plsc.parallel_loop(0, 128, unroll=4)
            def dot_n(j, _buf=buf, _r=r):
                flat = _r * 128 + j
                jv = jnp.full((_L,), j, jnp.int32)
                bv = plsc.load_gather(pkn_v, [jnp.full((_L,), flat, jnp.int32)])
                d = jnp.zeros((_L,), jnp.float32)
                for q in range(_Q):
                    cq = plsc.load_gather(c_all, [bv, iota + q * _L])
                    d = d + cq * _buf[j, pl.ds(q * _L, _L)]
                cum = plsc.cumsum(d)
                plsc.store_scatter(dblk, [jv], -cum, mask=m15)

            accum_dblk()
        return carry

    lax.fori_loop(0, _NROWS // 2, pair_body, 0)

    pltpu.sync_copy(acc_v, out_hbm.at[pl.ds(wid * _L, _L)])


def _finish_body(p_ref, o_ref):
    o_ref[...] = (-jnp.sum(p_ref[...]) * (1.0 / B)).reshape(1, 1)


_finish = pl.pallas_call(
    _finish_body,
    out_shape=jax.ShapeDtypeStruct((1, 1), jnp.float32),
)


def kernel(center, outside, negative, W_center, W_outside):
    c_i = center.astype(jnp.int32).reshape(B)
    o_i = outside.astype(jnp.int32).reshape(B)
    n_i = negative.astype(jnp.int32).reshape(B * K)

    gc = c_i.reshape(_NW, _CBLK, 128)
    go = o_i.reshape(_NW, _CBLK, 128)
    gco = jnp.concatenate([gc, go], axis=1).reshape(_NW * 8, 128)
    gn = n_i.reshape(B * K // 128, 128)

    pkn = (jnp.arange(B * K, dtype=jnp.int32) // K) % _BPW

    partials = _sc_loss(gco, gn, pkn, W_center, W_outside)
    return _finish(partials.reshape(_NW, _L))[0, 0]
